# baseline (speedup 1.0000x reference)
import jax
import jax.numpy as jnp
import numpy as np
from jax.experimental import pallas as pl

_B = 16
_BASE = 20
_A = 5
_C = 1
_H = 26
_W = 26
_HW = _H * _W
_MAXB = 10
_ANCHORS = np.array(
    [1.3221, 1.73145, 3.19275, 4.00944, 5.05587, 8.09892, 9.47112,
     4.84053, 11.2364, 10.0071], dtype=np.float32).reshape(_A, 2)
_THRESH = 0.6
_OBJECT_SCALE = 5.0


def _loss_kernel(out_ref, tgt_ref, loss_ref):
    b = pl.program_id(0)
    a = pl.program_id(1)

    xr = out_ref[:, 0, 0, :]
    yr = out_ref[:, 0, 1, :]
    wr = out_ref[:, 0, 2, :]
    hr = out_ref[:, 0, 3, :]
    cr = out_ref[:, 0, 4, :]
    lr = out_ref[:, 0, 5, :]

    sx = jax.nn.sigmoid(xr)
    sy = jax.nn.sigmoid(yr)
    sc = jax.nn.sigmoid(cr)

    s_idx = jax.lax.broadcasted_iota(jnp.int32, (1, _HW), 1)
    fi = (s_idx % _W).astype(jnp.float32)
    fj = (s_idx // _W).astype(jnp.float32)
    base_idx = jax.lax.broadcasted_iota(jnp.int32, (_BASE, 1), 0)

    anch_w = jnp.float32(0.0)
    anch_h = jnp.float32(0.0)
    for k in range(_A):
        anch_w = jnp.where(a == k, float(_ANCHORS[k, 0]), anch_w)
        anch_h = jnp.where(a == k, float(_ANCHORS[k, 1]), anch_h)

    px = sx + fi
    py = sy + fj
    pw = jnp.exp(wr) * anch_w
    ph = jnp.exp(hr) * anch_h
    p_l = px - 0.5 * pw
    p_r = px + 0.5 * pw
    p_t = py - 0.5 * ph
    p_b = py + 0.5 * ph
    p_area = pw * ph

    lab = jnp.clip((tgt_ref[0, 0] * _BASE).astype(jnp.int32), 0, _BASE - 1)
    gx = tgt_ref[0, 1] * _W
    gy = tgt_ref[0, 2] * _H
    gw = tgt_ref[0, 3] * _W
    gh = tgt_ref[0, 4] * _H
    valid = tgt_ref[0, 1] != 0.0

    best_n = jnp.zeros_like(lab)
    best_v = jnp.full(lab.shape, -1.0, jnp.float32)
    for k in range(_A):
        aw = float(_ANCHORS[k, 0])
        ah = float(_ANCHORS[k, 1])
        inter = jnp.minimum(gw, aw) * jnp.minimum(gh, ah)
        union = gw * gh + aw * ah - inter
        iou_a = inter / jnp.maximum(union, 1e-10)
        upd = iou_a > best_v
        best_n = jnp.where(upd, k, best_n)
        best_v = jnp.maximum(best_v, iou_a)

    gi = jnp.clip(gx.astype(jnp.int32), 0, _W - 1)
    gj = jnp.clip(gy.astype(jnp.int32), 0, _H - 1)
    tx_v = gx - gi.astype(jnp.float32)
    ty_v = gy - gj.astype(jnp.float32)
    aw_b = jnp.zeros_like(gw)
    ah_b = jnp.zeros_like(gh)
    for k in range(_A):
        sel = best_n == k
        aw_b = jnp.where(sel, float(_ANCHORS[k, 0]), aw_b)
        ah_b = jnp.where(sel, float(_ANCHORS[k, 1]), ah_b)
    tw_v = jnp.log(jnp.maximum(gw, 1e-8) / aw_b)
    th_v = jnp.log(jnp.maximum(gh, 1e-8) / ah_b)
    spos = gj * _W + gi

    g_l = gx - 0.5 * gw
    g_r = gx + 0.5 * gw
    g_t = gy - 0.5 * gh
    g_b = gy + 0.5 * gh
    g_area = gw * gh
    vb = valid & (best_n == a)

    shape = (_BASE, _HW)
    tx_c = jnp.full(shape, 0.5, jnp.float32)
    ty_c = jnp.full(shape, 0.5, jnp.float32)
    tw_c = jnp.zeros(shape, jnp.float32)
    th_c = jnp.zeros(shape, jnp.float32)
    inter_c = jnp.zeros(shape, jnp.float32)
    union_c = jnp.ones(shape, jnp.float32)
    anyobj = jnp.zeros(shape, jnp.bool_)
    wcode = jnp.zeros(shape, jnp.int32)

    for k in range(_MAXB):
        b_l = g_l[:, k].reshape(_BASE, 1)
        b_r = g_r[:, k].reshape(_BASE, 1)
        b_t = g_t[:, k].reshape(_BASE, 1)
        b_b = g_b[:, k].reshape(_BASE, 1)
        b_area = g_area[:, k].reshape(_BASE, 1)
        cw = jnp.maximum(
            jnp.minimum(p_r, b_r) - jnp.maximum(p_l, b_l), 0.0)
        ch = jnp.maximum(
            jnp.minimum(p_b, b_b) - jnp.maximum(p_t, b_t), 0.0)
        inter = cw * ch
        union = p_area + b_area - inter

        v_k = valid[:, k].reshape(_BASE, 1)
        anyobj = anyobj | (v_k & (inter > _THRESH * union))

        match = vb[:, k].reshape(_BASE, 1) \
            & (spos[:, k].reshape(_BASE, 1) == s_idx)
        tx_c = jnp.where(match, tx_v[:, k].reshape(_BASE, 1), tx_c)
        ty_c = jnp.where(match, ty_v[:, k].reshape(_BASE, 1), ty_c)
        tw_c = jnp.where(match, tw_v[:, k].reshape(_BASE, 1), tw_c)
        th_c = jnp.where(match, th_v[:, k].reshape(_BASE, 1), th_c)
        inter_c = jnp.where(match, inter, inter_c)
        union_c = jnp.where(match, union, union_c)
        code = (base_idx * _MAXB + (k + 1)) * 32 + lab[:, k].reshape(_BASE, 1)
        wcode = jnp.maximum(wcode, jnp.where(match, code, 0))

    hasm = wcode > 0
    tconf_c = jnp.where(
        hasm, inter_c / jnp.maximum(union_c, 1e-10), 0.0)
    conf_mask = jnp.where(
        hasm, _OBJECT_SCALE, jnp.where(anyobj, 0.0, 1.0))

    cell_sq = (sx - tx_c) ** 2 + (sy - ty_c) ** 2 \
        + (wr - tw_c) ** 2 + (hr - th_c) ** 2 \
        + (conf_mask * (sc - tconf_c)) ** 2
    s_both = jnp.sum(cell_sq)

    wc = jnp.max(wcode, axis=0)
    lab_sel = wc % 32
    has_cls = wc > 0
    m = jnp.max(lr, axis=0)
    lse = m + jnp.log(jnp.sum(jnp.exp(lr - m[None]), axis=0))
    logit_sel = jnp.sum(
        jnp.where(base_idx == lab_sel[None], lr, 0.0), axis=0)
    s_cls = jnp.sum(jnp.where(has_cls, lse - logit_sel, 0.0))

    total = 0.5 * s_both + s_cls

    row = jax.lax.broadcasted_iota(jnp.int32, (8, 128), 0)
    col = jax.lax.broadcasted_iota(jnp.int32, (8, 128), 1)
    contrib = jnp.where((row == 0) & (col == 0), total, 0.0)

    first = (b == 0) & (a == 0)

    @pl.when(first)
    def _init():
        loss_ref[...] = contrib

    @pl.when(jnp.logical_not(first))
    def _acc():
        loss_ref[...] += contrib


def kernel(output, target):
    n = _B * _BASE
    out5 = output.reshape(n, _A, 5 + _C, _HW)
    tgt_t = target.reshape(_B, _BASE, _MAXB, 5).transpose(0, 3, 1, 2)
    loss = pl.pallas_call(
        _loss_kernel,
        grid=(_B, _A),
        in_specs=[
            pl.BlockSpec((_BASE, 1, 5 + _C, _HW), lambda b, a: (b, a, 0, 0)),
            pl.BlockSpec((1, 5, _BASE, _MAXB), lambda b, a: (b, 0, 0, 0)),
        ],
        out_specs=pl.BlockSpec((8, 128), lambda b, a: (0, 0)),
        out_shape=jax.ShapeDtypeStruct((8, 128), jnp.float32),
    )(out5, tgt_t)
    return loss[0, 0]


# speedup vs baseline: 1.1010x; 1.1010x over previous
import jax
import jax.numpy as jnp
import numpy as np
from jax.experimental import pallas as pl

_B = 16
_BASE = 20
_A = 5
_C = 1
_H = 26
_W = 26
_HW = _H * _W
_MAXB = 10
_ANCHORS = np.array(
    [1.3221, 1.73145, 3.19275, 4.00944, 5.05587, 8.09892, 9.47112,
     4.84053, 11.2364, 10.0071], dtype=np.float32).reshape(_A, 2)
_THRESH = 0.6
_OBJECT_SCALE = 5.0


def _loss_kernel(out_ref, tgt_ref, loss_ref):
    b = pl.program_id(0)
    a = pl.program_id(1)

    xr = out_ref[:, 0, 0, :]
    yr = out_ref[:, 0, 1, :]
    wr = out_ref[:, 0, 2, :]
    hr = out_ref[:, 0, 3, :]
    cr = out_ref[:, 0, 4, :]
    lr = out_ref[:, 0, 5, :]

    sx = jax.nn.sigmoid(xr)
    sy = jax.nn.sigmoid(yr)
    sc = jax.nn.sigmoid(cr)

    s_idx = jax.lax.broadcasted_iota(jnp.int32, (1, _HW), 1)
    fi = (s_idx % _W).astype(jnp.float32)
    fj = (s_idx // _W).astype(jnp.float32)
    base_idx = jax.lax.broadcasted_iota(jnp.int32, (_BASE, 1), 0)

    anch_w = jnp.float32(0.0)
    anch_h = jnp.float32(0.0)
    for k in range(_A):
        anch_w = jnp.where(a == k, float(_ANCHORS[k, 0]), anch_w)
        anch_h = jnp.where(a == k, float(_ANCHORS[k, 1]), anch_h)

    px = sx + fi
    py = sy + fj
    pw = jnp.exp(wr) * anch_w
    ph = jnp.exp(hr) * anch_h
    p_l = px - 0.5 * pw
    p_r = px + 0.5 * pw
    p_t = py - 0.5 * ph
    p_b = py + 0.5 * ph
    p_area = pw * ph

    lab = jnp.clip((tgt_ref[0, 0] * _BASE).astype(jnp.int32), 0, _BASE - 1)
    gx = tgt_ref[0, 1] * _W
    gy = tgt_ref[0, 2] * _H
    gw = tgt_ref[0, 3] * _W
    gh = tgt_ref[0, 4] * _H
    valid = tgt_ref[0, 1] != 0.0

    best_n = jnp.zeros_like(lab)
    best_v = jnp.full(lab.shape, -1.0, jnp.float32)
    for k in range(_A):
        aw = float(_ANCHORS[k, 0])
        ah = float(_ANCHORS[k, 1])
        inter = jnp.minimum(gw, aw) * jnp.minimum(gh, ah)
        union = gw * gh + aw * ah - inter
        iou_a = inter / jnp.maximum(union, 1e-10)
        upd = iou_a > best_v
        best_n = jnp.where(upd, k, best_n)
        best_v = jnp.maximum(best_v, iou_a)

    gi = jnp.clip(gx.astype(jnp.int32), 0, _W - 1)
    gj = jnp.clip(gy.astype(jnp.int32), 0, _H - 1)
    tx_v = gx - gi.astype(jnp.float32)
    ty_v = gy - gj.astype(jnp.float32)
    aw_b = jnp.zeros_like(gw)
    ah_b = jnp.zeros_like(gh)
    for k in range(_A):
        sel = best_n == k
        aw_b = jnp.where(sel, float(_ANCHORS[k, 0]), aw_b)
        ah_b = jnp.where(sel, float(_ANCHORS[k, 1]), ah_b)
    tw_v = jnp.log(jnp.maximum(gw, 1e-8) / aw_b)
    th_v = jnp.log(jnp.maximum(gh, 1e-8) / ah_b)
    spos = gj * _W + gi

    g_l = gx - 0.5 * gw
    g_r = gx + 0.5 * gw
    g_t = gy - 0.5 * gh
    g_b = gy + 0.5 * gh
    g_area = gw * gh
    vb = valid & (best_n == a)

    shape = (_BASE, _HW)
    tx_c = jnp.full(shape, 0.5, jnp.float32)
    ty_c = jnp.full(shape, 0.5, jnp.float32)
    tw_c = jnp.zeros(shape, jnp.float32)
    th_c = jnp.zeros(shape, jnp.float32)
    tconf_c = jnp.zeros(shape, jnp.float32)
    max_iou = jnp.zeros(shape, jnp.float32)
    hasm = jnp.zeros(shape, jnp.float32)
    wcode = jnp.zeros(shape, jnp.int32)

    for k in range(_MAXB):
        b_l = g_l[:, k].reshape(_BASE, 1)
        b_r = g_r[:, k].reshape(_BASE, 1)
        b_t = g_t[:, k].reshape(_BASE, 1)
        b_b = g_b[:, k].reshape(_BASE, 1)
        b_area = g_area[:, k].reshape(_BASE, 1)
        cw = jnp.maximum(
            jnp.minimum(p_r, b_r) - jnp.maximum(p_l, b_l), 0.0)
        ch = jnp.maximum(
            jnp.minimum(p_b, b_b) - jnp.maximum(p_t, b_t), 0.0)
        inter = cw * ch
        iou = inter / jnp.maximum(p_area + b_area - inter, 1e-10)

        v_k = valid[:, k].reshape(_BASE, 1)
        max_iou = jnp.maximum(max_iou, jnp.where(v_k, iou, 0.0))

        match = vb[:, k].reshape(_BASE, 1) \
            & (spos[:, k].reshape(_BASE, 1) == s_idx)
        tx_c = jnp.where(match, tx_v[:, k].reshape(_BASE, 1), tx_c)
        ty_c = jnp.where(match, ty_v[:, k].reshape(_BASE, 1), ty_c)
        tw_c = jnp.where(match, tw_v[:, k].reshape(_BASE, 1), tw_c)
        th_c = jnp.where(match, th_v[:, k].reshape(_BASE, 1), th_c)
        tconf_c = jnp.where(match, iou, tconf_c)
        hasm = jnp.where(match, 1.0, hasm)
        code = (base_idx * _MAXB + (k + 1)) * 32 + lab[:, k].reshape(_BASE, 1)
        wcode = jnp.maximum(wcode, jnp.where(match, code, 0))

    conf_mask = jnp.where(
        hasm > 0.0, _OBJECT_SCALE, jnp.where(max_iou > _THRESH, 0.0, 1.0))

    cell_sq = (sx - tx_c) ** 2 + (sy - ty_c) ** 2 \
        + (wr - tw_c) ** 2 + (hr - th_c) ** 2 \
        + (conf_mask * (sc - tconf_c)) ** 2
    s_both = jnp.sum(cell_sq)

    wc = jnp.max(wcode, axis=0)
    lab_sel = wc % 32
    has_cls = wc > 0
    m = jnp.max(lr, axis=0)
    lse = m + jnp.log(jnp.sum(jnp.exp(lr - m[None]), axis=0))
    logit_sel = jnp.sum(
        jnp.where(base_idx == lab_sel[None], lr, 0.0), axis=0)
    s_cls = jnp.sum(jnp.where(has_cls, lse - logit_sel, 0.0))

    total = 0.5 * s_both + s_cls

    row = jax.lax.broadcasted_iota(jnp.int32, (8, 128), 0)
    col = jax.lax.broadcasted_iota(jnp.int32, (8, 128), 1)
    contrib = jnp.where((row == 0) & (col == 0), total, 0.0)

    first = (b == 0) & (a == 0)

    @pl.when(first)
    def _init():
        loss_ref[...] = contrib

    @pl.when(jnp.logical_not(first))
    def _acc():
        loss_ref[...] += contrib


def kernel(output, target):
    n = _B * _BASE
    out5 = output.reshape(n, _A, 5 + _C, _HW)
    tgt_t = target.reshape(_B, _BASE, _MAXB, 5).transpose(0, 3, 1, 2)
    loss = pl.pallas_call(
        _loss_kernel,
        grid=(_B, _A),
        in_specs=[
            pl.BlockSpec((_BASE, 1, 5 + _C, _HW), lambda b, a: (b, a, 0, 0)),
            pl.BlockSpec((1, 5, _BASE, _MAXB), lambda b, a: (b, 0, 0, 0)),
        ],
        out_specs=pl.BlockSpec((8, 128), lambda b, a: (0, 0)),
        out_shape=jax.ShapeDtypeStruct((8, 128), jnp.float32),
    )(out5, tgt_t)
    return loss[0, 0]


# speedup vs baseline: 1.1055x; 1.0041x over previous
import jax
import jax.numpy as jnp
import numpy as np
from jax.experimental import pallas as pl

_B = 16
_BASE = 20
_A = 5
_C = 1
_H = 26
_W = 26
_HW = _H * _W
_MAXB = 10
_ANCHORS = np.array(
    [1.3221, 1.73145, 3.19275, 4.00944, 5.05587, 8.09892, 9.47112,
     4.84053, 11.2364, 10.0071], dtype=np.float32).reshape(_A, 2)
_THRESH = 0.6
_OBJECT_SCALE = 5.0


def _loss_kernel(out_ref, tgt_ref, loss_ref):
    b = pl.program_id(0)
    a = pl.program_id(1)

    xr = out_ref[:, 0, 0, :]
    yr = out_ref[:, 0, 1, :]
    wr = out_ref[:, 0, 2, :]
    hr = out_ref[:, 0, 3, :]
    cr = out_ref[:, 0, 4, :]
    lr = out_ref[:, 0, 5, :]

    sx = jax.nn.sigmoid(xr)
    sy = jax.nn.sigmoid(yr)
    sc = jax.nn.sigmoid(cr)

    s_idx = jax.lax.broadcasted_iota(jnp.int32, (1, _HW), 1)
    fi = (s_idx % _W).astype(jnp.float32)
    fj = (s_idx // _W).astype(jnp.float32)
    base_idx = jax.lax.broadcasted_iota(jnp.int32, (_BASE, 1), 0)

    anch_w = jnp.float32(0.0)
    anch_h = jnp.float32(0.0)
    for k in range(_A):
        anch_w = jnp.where(a == k, float(_ANCHORS[k, 0]), anch_w)
        anch_h = jnp.where(a == k, float(_ANCHORS[k, 1]), anch_h)

    px = sx + fi
    py = sy + fj
    pw = jnp.exp(wr) * anch_w
    ph = jnp.exp(hr) * anch_h
    p_l = px - 0.5 * pw
    p_r = px + 0.5 * pw
    p_t = py - 0.5 * ph
    p_b = py + 0.5 * ph
    p_area = pw * ph

    lab = jnp.clip((tgt_ref[0, 0] * _BASE).astype(jnp.int32), 0, _BASE - 1)
    gx = tgt_ref[0, 1] * _W
    gy = tgt_ref[0, 2] * _H
    gw = tgt_ref[0, 3] * _W
    gh = tgt_ref[0, 4] * _H
    valid = tgt_ref[0, 1] != 0.0

    best_n = jnp.zeros_like(lab)
    best_v = jnp.full(lab.shape, -1.0, jnp.float32)
    for k in range(_A):
        aw = float(_ANCHORS[k, 0])
        ah = float(_ANCHORS[k, 1])
        inter = jnp.minimum(gw, aw) * jnp.minimum(gh, ah)
        union = gw * gh + aw * ah - inter
        iou_a = inter / jnp.maximum(union, 1e-10)
        upd = iou_a > best_v
        best_n = jnp.where(upd, k, best_n)
        best_v = jnp.maximum(best_v, iou_a)

    gi = jnp.clip(gx.astype(jnp.int32), 0, _W - 1)
    gj = jnp.clip(gy.astype(jnp.int32), 0, _H - 1)
    tx_v = gx - gi.astype(jnp.float32)
    ty_v = gy - gj.astype(jnp.float32)
    aw_b = jnp.zeros_like(gw)
    ah_b = jnp.zeros_like(gh)
    for k in range(_A):
        sel = best_n == k
        aw_b = jnp.where(sel, float(_ANCHORS[k, 0]), aw_b)
        ah_b = jnp.where(sel, float(_ANCHORS[k, 1]), ah_b)
    tw_v = jnp.log(jnp.maximum(gw, 1e-8) / aw_b)
    th_v = jnp.log(jnp.maximum(gh, 1e-8) / ah_b)
    spos = gj * _W + gi

    g_l = gx - 0.5 * gw
    g_r = gx + 0.5 * gw
    g_t = gy - 0.5 * gh
    g_b = gy + 0.5 * gh
    g_area = gw * gh
    vb = valid & (best_n == a)
    k_iota = jax.lax.broadcasted_iota(jnp.int32, (1, _MAXB), 1)
    codes = (base_idx * _MAXB + (k_iota + 1)) * 32 + lab

    shape = (_BASE, _HW)
    tx_c = jnp.full(shape, 0.5, jnp.float32)
    ty_c = jnp.full(shape, 0.5, jnp.float32)
    tw_c = jnp.zeros(shape, jnp.float32)
    th_c = jnp.zeros(shape, jnp.float32)
    tconf_c = jnp.zeros(shape, jnp.float32)
    max_iou = jnp.zeros(shape, jnp.float32)
    hasm = jnp.zeros(shape, jnp.float32)
    wcode = jnp.zeros(shape, jnp.int32)

    for k in range(_MAXB):
        b_l = g_l[:, k].reshape(_BASE, 1)
        b_r = g_r[:, k].reshape(_BASE, 1)
        b_t = g_t[:, k].reshape(_BASE, 1)
        b_b = g_b[:, k].reshape(_BASE, 1)
        b_area = g_area[:, k].reshape(_BASE, 1)
        cw = jnp.maximum(
            jnp.minimum(p_r, b_r) - jnp.maximum(p_l, b_l), 0.0)
        ch = jnp.maximum(
            jnp.minimum(p_b, b_b) - jnp.maximum(p_t, b_t), 0.0)
        inter = cw * ch
        iou = inter / jnp.maximum(p_area + b_area - inter, 1e-10)

        v_k = valid[:, k].reshape(_BASE, 1)
        max_iou = jnp.maximum(max_iou, jnp.where(v_k, iou, 0.0))

        match = vb[:, k].reshape(_BASE, 1) \
            & (spos[:, k].reshape(_BASE, 1) == s_idx)
        tx_c = jnp.where(match, tx_v[:, k].reshape(_BASE, 1), tx_c)
        ty_c = jnp.where(match, ty_v[:, k].reshape(_BASE, 1), ty_c)
        tw_c = jnp.where(match, tw_v[:, k].reshape(_BASE, 1), tw_c)
        th_c = jnp.where(match, th_v[:, k].reshape(_BASE, 1), th_c)
        tconf_c = jnp.where(match, iou, tconf_c)
        hasm = jnp.where(match, 1.0, hasm)
        code = codes[:, k].reshape(_BASE, 1)
        wcode = jnp.maximum(wcode, jnp.where(match, code, 0))

    conf_mask = jnp.where(
        hasm > 0.0, _OBJECT_SCALE,
        jnp.where(max_iou > _THRESH, 0.0, 1.0))

    cell_sq = (sx - tx_c) ** 2 + (sy - ty_c) ** 2 \
        + (wr - tw_c) ** 2 + (hr - th_c) ** 2 \
        + (conf_mask * (sc - tconf_c)) ** 2
    s_both = jnp.sum(cell_sq)

    wc = jnp.max(wcode, axis=0)
    lab_sel = wc % 32
    has_cls = wc > 0
    m = jnp.max(lr, axis=0)
    lse = m + jnp.log(jnp.sum(jnp.exp(lr - m[None]), axis=0))
    logit_sel = jnp.sum(
        jnp.where(base_idx == lab_sel[None], lr, 0.0), axis=0)
    s_cls = jnp.sum(jnp.where(has_cls, lse - logit_sel, 0.0))

    total = 0.5 * s_both + s_cls

    row = jax.lax.broadcasted_iota(jnp.int32, (8, 128), 0)
    col = jax.lax.broadcasted_iota(jnp.int32, (8, 128), 1)
    contrib = jnp.where((row == 0) & (col == 0), total, 0.0)

    first = (b == 0) & (a == 0)

    @pl.when(first)
    def _init():
        loss_ref[...] = contrib

    @pl.when(jnp.logical_not(first))
    def _acc():
        loss_ref[...] += contrib


def kernel(output, target):
    n = _B * _BASE
    out5 = output.reshape(n, _A, 5 + _C, _HW)
    tgt_t = target.reshape(_B, _BASE, _MAXB, 5).transpose(0, 3, 1, 2)
    loss = pl.pallas_call(
        _loss_kernel,
        grid=(_B, _A),
        in_specs=[
            pl.BlockSpec((_BASE, 1, 5 + _C, _HW), lambda b, a: (b, a, 0, 0)),
            pl.BlockSpec((1, 5, _BASE, _MAXB), lambda b, a: (b, 0, 0, 0)),
        ],
        out_specs=pl.BlockSpec((8, 128), lambda b, a: (0, 0)),
        out_shape=jax.ShapeDtypeStruct((8, 128), jnp.float32),
    )(out5, tgt_t)
    return loss[0, 0]


# speedup vs baseline: 1.1417x; 1.0328x over previous
import jax
import jax.numpy as jnp
import numpy as np
from jax.experimental import pallas as pl

_B = 16
_BASE = 20
_A = 5
_C = 1
_H = 26
_W = 26
_HW = _H * _W
_MAXB = 10
_ANCHORS = np.array(
    [1.3221, 1.73145, 3.19275, 4.00944, 5.05587, 8.09892, 9.47112,
     4.84053, 11.2364, 10.0071], dtype=np.float32).reshape(_A, 2)
_THRESH = 0.6
_OBJECT_SCALE = 5.0


def _loss_kernel(out_ref, tgt_ref, loss_ref):
    b = pl.program_id(0)
    a = pl.program_id(1)

    xr = out_ref[:, 0, 0, :]
    yr = out_ref[:, 0, 1, :]
    wr = out_ref[:, 0, 2, :]
    hr = out_ref[:, 0, 3, :]
    cr = out_ref[:, 0, 4, :]
    lr = out_ref[:, 0, 5, :]

    sx = jax.nn.sigmoid(xr)
    sy = jax.nn.sigmoid(yr)
    sc = jax.nn.sigmoid(cr)

    s_idx = jax.lax.broadcasted_iota(jnp.int32, (1, _HW), 1)
    fi = (s_idx % _W).astype(jnp.float32)
    fj = (s_idx // _W).astype(jnp.float32)
    base_idx = jax.lax.broadcasted_iota(jnp.int32, (_BASE, 1), 0)

    anch_w = jnp.float32(0.0)
    anch_h = jnp.float32(0.0)
    for k in range(_A):
        anch_w = jnp.where(a == k, float(_ANCHORS[k, 0]), anch_w)
        anch_h = jnp.where(a == k, float(_ANCHORS[k, 1]), anch_h)

    px = sx + fi
    py = sy + fj
    pw = jnp.exp(wr) * anch_w
    ph = jnp.exp(hr) * anch_h
    p_l = px - 0.5 * pw
    p_r = px + 0.5 * pw
    p_t = py - 0.5 * ph
    p_b = py + 0.5 * ph
    p_area = pw * ph

    lab = jnp.clip((tgt_ref[0, 0] * _BASE).astype(jnp.int32), 0, _BASE - 1)
    gx = tgt_ref[0, 1] * _W
    gy = tgt_ref[0, 2] * _H
    gw = tgt_ref[0, 3] * _W
    gh = tgt_ref[0, 4] * _H
    valid = tgt_ref[0, 1] != 0.0

    best_n = jnp.zeros_like(lab)
    best_v = jnp.full(lab.shape, -1.0, jnp.float32)
    for k in range(_A):
        aw = float(_ANCHORS[k, 0])
        ah = float(_ANCHORS[k, 1])
        inter = jnp.minimum(gw, aw) * jnp.minimum(gh, ah)
        union = gw * gh + aw * ah - inter
        iou_a = inter / jnp.maximum(union, 1e-10)
        upd = iou_a > best_v
        best_n = jnp.where(upd, k, best_n)
        best_v = jnp.maximum(best_v, iou_a)

    gi = jnp.clip(gx.astype(jnp.int32), 0, _W - 1)
    gj = jnp.clip(gy.astype(jnp.int32), 0, _H - 1)
    tx_v = gx - gi.astype(jnp.float32)
    ty_v = gy - gj.astype(jnp.float32)
    aw_b = jnp.zeros_like(gw)
    ah_b = jnp.zeros_like(gh)
    for k in range(_A):
        sel = best_n == k
        aw_b = jnp.where(sel, float(_ANCHORS[k, 0]), aw_b)
        ah_b = jnp.where(sel, float(_ANCHORS[k, 1]), ah_b)
    tw_v = jnp.log(jnp.maximum(gw, 1e-8) / aw_b)
    th_v = jnp.log(jnp.maximum(gh, 1e-8) / ah_b)
    spos = gj * _W + gi

    g_l = jnp.where(valid, gx - 0.5 * gw, -1e6)
    g_r = jnp.where(valid, gx + 0.5 * gw, -1e6)
    g_t = jnp.where(valid, gy - 0.5 * gh, -1e6)
    g_b = jnp.where(valid, gy + 0.5 * gh, -1e6)
    g_area = gw * gh
    vb = valid & (best_n == a)
    sposm = jnp.where(vb, spos, -1)
    k_iota = jax.lax.broadcasted_iota(jnp.int32, (1, _MAXB), 1)
    codes = (base_idx * _MAXB + (k_iota + 1)) * 32 + lab

    shape = (_BASE, _HW)
    tx_c = jnp.full(shape, 0.5, jnp.float32)
    ty_c = jnp.full(shape, 0.5, jnp.float32)
    tw_c = jnp.zeros(shape, jnp.float32)
    th_c = jnp.zeros(shape, jnp.float32)
    tconf_c = jnp.zeros(shape, jnp.float32)
    max_iou = jnp.zeros(shape, jnp.float32)
    wcode = jnp.zeros(shape, jnp.int32)

    for k in range(_MAXB):
        b_l = g_l[:, k].reshape(_BASE, 1)
        b_r = g_r[:, k].reshape(_BASE, 1)
        b_t = g_t[:, k].reshape(_BASE, 1)
        b_b = g_b[:, k].reshape(_BASE, 1)
        b_area = g_area[:, k].reshape(_BASE, 1)
        cw = jnp.maximum(
            jnp.minimum(p_r, b_r) - jnp.maximum(p_l, b_l), 0.0)
        ch = jnp.maximum(
            jnp.minimum(p_b, b_b) - jnp.maximum(p_t, b_t), 0.0)
        inter = cw * ch
        iou = inter / jnp.maximum(p_area + b_area - inter, 1e-10)

        max_iou = jnp.maximum(max_iou, iou)

        match = sposm[:, k].reshape(_BASE, 1) == s_idx
        tx_c = jnp.where(match, tx_v[:, k].reshape(_BASE, 1), tx_c)
        ty_c = jnp.where(match, ty_v[:, k].reshape(_BASE, 1), ty_c)
        tw_c = jnp.where(match, tw_v[:, k].reshape(_BASE, 1), tw_c)
        th_c = jnp.where(match, th_v[:, k].reshape(_BASE, 1), th_c)
        tconf_c = jnp.where(match, iou, tconf_c)
        code = codes[:, k].reshape(_BASE, 1)
        wcode = jnp.maximum(wcode, jnp.where(match, code, 0))

    conf_mask = jnp.where(
        wcode > 0, _OBJECT_SCALE,
        jnp.where(max_iou > _THRESH, 0.0, 1.0))

    cell_sq = (sx - tx_c) ** 2 + (sy - ty_c) ** 2 \
        + (wr - tw_c) ** 2 + (hr - th_c) ** 2 \
        + (conf_mask * (sc - tconf_c)) ** 2
    s_both = jnp.sum(cell_sq)

    wc = jnp.max(wcode, axis=0)
    lab_sel = wc % 32
    has_cls = wc > 0
    m = jnp.max(lr, axis=0)
    lse = m + jnp.log(jnp.sum(jnp.exp(lr - m[None]), axis=0))
    logit_sel = jnp.sum(
        jnp.where(base_idx == lab_sel[None], lr, 0.0), axis=0)
    s_cls = jnp.sum(jnp.where(has_cls, lse - logit_sel, 0.0))

    total = 0.5 * s_both + s_cls

    row = jax.lax.broadcasted_iota(jnp.int32, (8, 128), 0)
    col = jax.lax.broadcasted_iota(jnp.int32, (8, 128), 1)
    contrib = jnp.where((row == 0) & (col == 0), total, 0.0)

    first = (b == 0) & (a == 0)

    @pl.when(first)
    def _init():
        loss_ref[...] = contrib

    @pl.when(jnp.logical_not(first))
    def _acc():
        loss_ref[...] += contrib


def kernel(output, target):
    n = _B * _BASE
    out5 = output.reshape(n, _A, 5 + _C, _HW)
    tgt_t = target.reshape(_B, _BASE, _MAXB, 5).transpose(0, 3, 1, 2)
    loss = pl.pallas_call(
        _loss_kernel,
        grid=(_B, _A),
        in_specs=[
            pl.BlockSpec((_BASE, 1, 5 + _C, _HW), lambda b, a: (b, a, 0, 0)),
            pl.BlockSpec((1, 5, _BASE, _MAXB), lambda b, a: (b, 0, 0, 0)),
        ],
        out_specs=pl.BlockSpec((8, 128), lambda b, a: (0, 0)),
        out_shape=jax.ShapeDtypeStruct((8, 128), jnp.float32),
    )(out5, tgt_t)
    return loss[0, 0]


# speedup vs baseline: 1.2626x; 1.1058x over previous
import jax
import jax.numpy as jnp
import numpy as np
from jax.experimental import pallas as pl

_B = 16
_BASE = 20
_A = 5
_C = 1
_H = 26
_W = 26
_HW = _H * _W
_MAXB = 10
_ANCHORS = np.array(
    [1.3221, 1.73145, 3.19275, 4.00944, 5.05587, 8.09892, 9.47112,
     4.84053, 11.2364, 10.0071], dtype=np.float32).reshape(_A, 2)
_THRESH = 0.6
_OBJECT_SCALE = 5.0


def _loss_kernel(out_ref, tgt_ref, loss_ref):
    b = pl.program_id(0)

    s_idx = jax.lax.broadcasted_iota(jnp.int32, (1, _HW), 1)
    fi = (s_idx % _W).astype(jnp.float32)
    fj = (s_idx // _W).astype(jnp.float32)
    base_idx = jax.lax.broadcasted_iota(jnp.int32, (_BASE, 1), 0)

    lab = jnp.clip((tgt_ref[0, 0] * _BASE).astype(jnp.int32), 0, _BASE - 1)
    gx = tgt_ref[0, 1] * _W
    gy = tgt_ref[0, 2] * _H
    gw = tgt_ref[0, 3] * _W
    gh = tgt_ref[0, 4] * _H
    valid = tgt_ref[0, 1] != 0.0

    best_n = jnp.zeros_like(lab)
    best_v = jnp.full(lab.shape, -1.0, jnp.float32)
    for k in range(_A):
        aw = float(_ANCHORS[k, 0])
        ah = float(_ANCHORS[k, 1])
        inter = jnp.minimum(gw, aw) * jnp.minimum(gh, ah)
        union = gw * gh + aw * ah - inter
        iou_a = inter / jnp.maximum(union, 1e-10)
        upd = iou_a > best_v
        best_n = jnp.where(upd, k, best_n)
        best_v = jnp.maximum(best_v, iou_a)

    gi = jnp.clip(gx.astype(jnp.int32), 0, _W - 1)
    gj = jnp.clip(gy.astype(jnp.int32), 0, _H - 1)
    tx_v = gx - gi.astype(jnp.float32)
    ty_v = gy - gj.astype(jnp.float32)
    aw_b = jnp.zeros_like(gw)
    ah_b = jnp.zeros_like(gh)
    for k in range(_A):
        sel = best_n == k
        aw_b = jnp.where(sel, float(_ANCHORS[k, 0]), aw_b)
        ah_b = jnp.where(sel, float(_ANCHORS[k, 1]), ah_b)
    tw_v = jnp.log(jnp.maximum(gw, 1e-8) / aw_b)
    th_v = jnp.log(jnp.maximum(gh, 1e-8) / ah_b)
    spos = gj * _W + gi

    g_l = jnp.where(valid, gx - 0.5 * gw, -1e6)
    g_r = jnp.where(valid, gx + 0.5 * gw, -1e6)
    g_t = jnp.where(valid, gy - 0.5 * gh, -1e6)
    g_b = jnp.where(valid, gy + 0.5 * gh, -1e6)
    g_area = gw * gh
    k_iota = jax.lax.broadcasted_iota(jnp.int32, (1, _MAXB), 1)
    codes = (base_idx * _MAXB + (k_iota + 1)) * 32 + lab

    shape = (_BASE, _HW)
    total = jnp.float32(0.0)

    for a in range(_A):
        xr = out_ref[:, a, 0, :]
        yr = out_ref[:, a, 1, :]
        wr = out_ref[:, a, 2, :]
        hr = out_ref[:, a, 3, :]
        cr = out_ref[:, a, 4, :]
        lr = out_ref[:, a, 5, :]

        sx = jax.nn.sigmoid(xr)
        sy = jax.nn.sigmoid(yr)
        sc = jax.nn.sigmoid(cr)

        px = sx + fi
        py = sy + fj
        pw = jnp.exp(wr) * float(_ANCHORS[a, 0])
        ph = jnp.exp(hr) * float(_ANCHORS[a, 1])
        p_l = px - 0.5 * pw
        p_r = px + 0.5 * pw
        p_t = py - 0.5 * ph
        p_b = py + 0.5 * ph
        p_area = pw * ph

        sposm = jnp.where(valid & (best_n == a), spos, -1)

        tx_c = jnp.full(shape, 0.5, jnp.float32)
        ty_c = jnp.full(shape, 0.5, jnp.float32)
        tw_c = jnp.zeros(shape, jnp.float32)
        th_c = jnp.zeros(shape, jnp.float32)
        tconf_c = jnp.zeros(shape, jnp.float32)
        max_iou = jnp.zeros(shape, jnp.float32)
        wcode = jnp.zeros(shape, jnp.int32)

        for k in range(_MAXB):
            b_l = g_l[:, k].reshape(_BASE, 1)
            b_r = g_r[:, k].reshape(_BASE, 1)
            b_t = g_t[:, k].reshape(_BASE, 1)
            b_b = g_b[:, k].reshape(_BASE, 1)
            b_area = g_area[:, k].reshape(_BASE, 1)
            cw = jnp.maximum(
                jnp.minimum(p_r, b_r) - jnp.maximum(p_l, b_l), 0.0)
            ch = jnp.maximum(
                jnp.minimum(p_b, b_b) - jnp.maximum(p_t, b_t), 0.0)
            inter = cw * ch
            iou = inter / jnp.maximum(p_area + b_area - inter, 1e-10)

            max_iou = jnp.maximum(max_iou, iou)

            match = sposm[:, k].reshape(_BASE, 1) == s_idx
            tx_c = jnp.where(match, tx_v[:, k].reshape(_BASE, 1), tx_c)
            ty_c = jnp.where(match, ty_v[:, k].reshape(_BASE, 1), ty_c)
            tw_c = jnp.where(match, tw_v[:, k].reshape(_BASE, 1), tw_c)
            th_c = jnp.where(match, th_v[:, k].reshape(_BASE, 1), th_c)
            tconf_c = jnp.where(match, iou, tconf_c)
            code = codes[:, k].reshape(_BASE, 1)
            wcode = jnp.maximum(wcode, jnp.where(match, code, 0))

        conf_mask = jnp.where(
            wcode > 0, _OBJECT_SCALE,
            jnp.where(max_iou > _THRESH, 0.0, 1.0))

        cell_sq = (sx - tx_c) ** 2 + (sy - ty_c) ** 2 \
            + (wr - tw_c) ** 2 + (hr - th_c) ** 2 \
            + (conf_mask * (sc - tconf_c)) ** 2
        s_both = jnp.sum(cell_sq)

        wc = jnp.max(wcode, axis=0)
        lab_sel = wc % 32
        has_cls = wc > 0
        m = jnp.max(lr, axis=0)
        lse = m + jnp.log(jnp.sum(jnp.exp(lr - m[None]), axis=0))
        logit_sel = jnp.sum(
            jnp.where(base_idx == lab_sel[None], lr, 0.0), axis=0)
        s_cls = jnp.sum(jnp.where(has_cls, lse - logit_sel, 0.0))

        total = total + 0.5 * s_both + s_cls

    row = jax.lax.broadcasted_iota(jnp.int32, (8, 128), 0)
    col = jax.lax.broadcasted_iota(jnp.int32, (8, 128), 1)
    contrib = jnp.where((row == 0) & (col == 0), total, 0.0)

    @pl.when(b == 0)
    def _init():
        loss_ref[...] = contrib

    @pl.when(b != 0)
    def _acc():
        loss_ref[...] += contrib


def kernel(output, target):
    n = _B * _BASE
    out5 = output.reshape(n, _A, 5 + _C, _HW)
    tgt_t = target.reshape(_B, _BASE, _MAXB, 5).transpose(0, 3, 1, 2)
    loss = pl.pallas_call(
        _loss_kernel,
        grid=(_B,),
        in_specs=[
            pl.BlockSpec((_BASE, _A, 5 + _C, _HW), lambda b: (b, 0, 0, 0)),
            pl.BlockSpec((1, 5, _BASE, _MAXB), lambda b: (b, 0, 0, 0)),
        ],
        out_specs=pl.BlockSpec((8, 128), lambda b: (0, 0)),
        out_shape=jax.ShapeDtypeStruct((8, 128), jnp.float32),
    )(out5, tgt_t)
    return loss[0, 0]


# speedup vs baseline: 1.2811x; 1.0147x over previous
import jax
import jax.numpy as jnp
import numpy as np
from jax.experimental import pallas as pl

_B = 16
_BASE = 20
_A = 5
_C = 1
_H = 26
_W = 26
_HW = _H * _W
_MAXB = 10
_ANCHORS = np.array(
    [1.3221, 1.73145, 3.19275, 4.00944, 5.05587, 8.09892, 9.47112,
     4.84053, 11.2364, 10.0071], dtype=np.float32).reshape(_A, 2)
_THRESH = 0.6
_OBJECT_SCALE = 5.0


_GB = 2
_R = _GB * _BASE


def _loss_kernel(out_ref, tgt_ref, loss_ref):
    b = pl.program_id(0)

    s_idx = jax.lax.broadcasted_iota(jnp.int32, (1, _HW), 1)
    fi = (s_idx % _W).astype(jnp.float32)
    fj = (s_idx // _W).astype(jnp.float32)
    row_idx = jax.lax.broadcasted_iota(jnp.int32, (_R, 1), 0)
    base_idx = row_idx % _BASE

    lab = jnp.clip((tgt_ref[0, 0] * _BASE).astype(jnp.int32), 0, _BASE - 1)
    gx = tgt_ref[0, 1] * _W
    gy = tgt_ref[0, 2] * _H
    gw = tgt_ref[0, 3] * _W
    gh = tgt_ref[0, 4] * _H
    valid = tgt_ref[0, 1] != 0.0

    best_n = jnp.zeros_like(lab)
    best_v = jnp.full(lab.shape, -1.0, jnp.float32)
    for k in range(_A):
        aw = float(_ANCHORS[k, 0])
        ah = float(_ANCHORS[k, 1])
        inter = jnp.minimum(gw, aw) * jnp.minimum(gh, ah)
        union = gw * gh + aw * ah - inter
        iou_a = inter / jnp.maximum(union, 1e-10)
        upd = iou_a > best_v
        best_n = jnp.where(upd, k, best_n)
        best_v = jnp.maximum(best_v, iou_a)

    gi = jnp.clip(gx.astype(jnp.int32), 0, _W - 1)
    gj = jnp.clip(gy.astype(jnp.int32), 0, _H - 1)
    tx_v = gx - gi.astype(jnp.float32)
    ty_v = gy - gj.astype(jnp.float32)
    aw_b = jnp.zeros_like(gw)
    ah_b = jnp.zeros_like(gh)
    for k in range(_A):
        sel = best_n == k
        aw_b = jnp.where(sel, float(_ANCHORS[k, 0]), aw_b)
        ah_b = jnp.where(sel, float(_ANCHORS[k, 1]), ah_b)
    tw_v = jnp.log(jnp.maximum(gw, 1e-8) / aw_b)
    th_v = jnp.log(jnp.maximum(gh, 1e-8) / ah_b)
    spos = gj * _W + gi

    g_l = jnp.where(valid, gx - 0.5 * gw, -1e6)
    g_r = jnp.where(valid, gx + 0.5 * gw, -1e6)
    g_t = jnp.where(valid, gy - 0.5 * gh, -1e6)
    g_b = jnp.where(valid, gy + 0.5 * gh, -1e6)
    g_area = gw * gh
    k_iota = jax.lax.broadcasted_iota(jnp.int32, (1, _MAXB), 1)
    codes = (base_idx * _MAXB + (k_iota + 1)) * 32 + lab

    shape = (_R, _HW)
    total = jnp.float32(0.0)

    for a in range(_A):
        xr = out_ref[:, a, 0, :]
        yr = out_ref[:, a, 1, :]
        wr = out_ref[:, a, 2, :]
        hr = out_ref[:, a, 3, :]
        cr = out_ref[:, a, 4, :]
        lr = out_ref[:, a, 5, :]

        sx = jax.nn.sigmoid(xr)
        sy = jax.nn.sigmoid(yr)
        sc = jax.nn.sigmoid(cr)

        px = sx + fi
        py = sy + fj
        pw = jnp.exp(wr) * float(_ANCHORS[a, 0])
        ph = jnp.exp(hr) * float(_ANCHORS[a, 1])
        p_l = px - 0.5 * pw
        p_r = px + 0.5 * pw
        p_t = py - 0.5 * ph
        p_b = py + 0.5 * ph
        p_area = pw * ph

        sposm = jnp.where(valid & (best_n == a), spos, -1)

        tx_c = jnp.full(shape, 0.5, jnp.float32)
        ty_c = jnp.full(shape, 0.5, jnp.float32)
        tw_c = jnp.zeros(shape, jnp.float32)
        th_c = jnp.zeros(shape, jnp.float32)
        tconf_c = jnp.zeros(shape, jnp.float32)
        max_iou = jnp.zeros(shape, jnp.float32)
        wcode = jnp.zeros(shape, jnp.int32)

        for k in range(_MAXB):
            b_l = g_l[:, k].reshape(_R, 1)
            b_r = g_r[:, k].reshape(_R, 1)
            b_t = g_t[:, k].reshape(_R, 1)
            b_b = g_b[:, k].reshape(_R, 1)
            b_area = g_area[:, k].reshape(_R, 1)
            cw = jnp.maximum(
                jnp.minimum(p_r, b_r) - jnp.maximum(p_l, b_l), 0.0)
            ch = jnp.maximum(
                jnp.minimum(p_b, b_b) - jnp.maximum(p_t, b_t), 0.0)
            inter = cw * ch
            iou = inter / jnp.maximum(p_area + b_area - inter, 1e-10)

            max_iou = jnp.maximum(max_iou, iou)

            match = sposm[:, k].reshape(_R, 1) == s_idx
            tx_c = jnp.where(match, tx_v[:, k].reshape(_R, 1), tx_c)
            ty_c = jnp.where(match, ty_v[:, k].reshape(_R, 1), ty_c)
            tw_c = jnp.where(match, tw_v[:, k].reshape(_R, 1), tw_c)
            th_c = jnp.where(match, th_v[:, k].reshape(_R, 1), th_c)
            tconf_c = jnp.where(match, iou, tconf_c)
            code = codes[:, k].reshape(_R, 1)
            wcode = jnp.maximum(wcode, jnp.where(match, code, 0))

        conf_mask = jnp.where(
            wcode > 0, _OBJECT_SCALE,
            jnp.where(max_iou > _THRESH, 0.0, 1.0))

        cell_sq = (sx - tx_c) ** 2 + (sy - ty_c) ** 2 \
            + (wr - tw_c) ** 2 + (hr - th_c) ** 2 \
            + (conf_mask * (sc - tconf_c)) ** 2
        s_both = jnp.sum(cell_sq)

        s_cls = jnp.float32(0.0)
        for g2 in range(_GB):
            lo = g2 * _BASE
            hi = lo + _BASE
            wch = wcode[lo:hi]
            lrh = lr[lo:hi]
            bih = base_idx[lo:hi]
            wc = jnp.max(wch, axis=0)
            lab_sel = wc % 32
            has_cls = wc > 0
            m = jnp.max(lrh, axis=0)
            lse = m + jnp.log(jnp.sum(jnp.exp(lrh - m[None]), axis=0))
            logit_sel = jnp.sum(
                jnp.where(bih == lab_sel[None], lrh, 0.0), axis=0)
            s_cls = s_cls + jnp.sum(
                jnp.where(has_cls, lse - logit_sel, 0.0))

        total = total + 0.5 * s_both + s_cls

    row = jax.lax.broadcasted_iota(jnp.int32, (8, 128), 0)
    col = jax.lax.broadcasted_iota(jnp.int32, (8, 128), 1)
    contrib = jnp.where((row == 0) & (col == 0), total, 0.0)

    @pl.when(b == 0)
    def _init():
        loss_ref[...] = contrib

    @pl.when(b != 0)
    def _acc():
        loss_ref[...] += contrib


def kernel(output, target):
    n = _B * _BASE
    out5 = output.reshape(n, _A, 5 + _C, _HW)
    tgt_t = target.reshape(_B // _GB, _GB, _BASE, _MAXB, 5) \
        .transpose(0, 4, 1, 2, 3).reshape(_B // _GB, 5, _R, _MAXB)
    loss = pl.pallas_call(
        _loss_kernel,
        grid=(_B // _GB,),
        in_specs=[
            pl.BlockSpec((_R, _A, 5 + _C, _HW), lambda b: (b, 0, 0, 0)),
            pl.BlockSpec((1, 5, _R, _MAXB), lambda b: (b, 0, 0, 0)),
        ],
        out_specs=pl.BlockSpec((8, 128), lambda b: (0, 0)),
        out_shape=jax.ShapeDtypeStruct((8, 128), jnp.float32),
    )(out5, tgt_t)
    return loss[0, 0]
